# baseline (speedup 1.0000x reference)
import functools

import jax
import jax.numpy as jnp
from jax import lax
from jax.experimental import pallas as pl
from jax.experimental.pallas import tpu as pltpu
from jax.experimental.pallas import tpu_sc as plsc

_D = 64
_CHUNK = 128
_LANES = 16


@functools.lru_cache(maxsize=None)
def _make_sc_gather(num_workers, n_chunks, rows_per_worker, total_rows, scale):
    mesh = plsc.VectorSubcoreMesh(core_axis_name="c", subcore_axis_name="s")
    num_cores = 2

    @functools.partial(
        pl.kernel,
        out_type=jax.ShapeDtypeStruct((total_rows, _D), jnp.float32),
        mesh=mesh,
        scratch_types=[
            pltpu.VMEM((n_chunks, _CHUNK), jnp.int32),
            pltpu.VMEM((_CHUNK, _D), jnp.float32),
            pltpu.SemaphoreType.DMA,
        ],
        compiler_params=pltpu.CompilerParams(use_tc_tiling_on_sc=False),
    )
    def sc_gather(ids_hbm, table_hbm, out_hbm, idx_v, buf, sem):
        wid = lax.axis_index("s") * num_cores + lax.axis_index("c")
        base = wid * rows_per_worker
        pltpu.sync_copy(ids_hbm.at[wid], idx_v)

        def chunk_body(j, carry):
            pltpu.async_copy(table_hbm.at[idx_v.at[j]], buf, sem).wait()

            def scale_row(i, c):
                for col in range(_D // _LANES):
                    sl = pl.ds(col * _LANES, _LANES)
                    buf[i, sl] = buf[i, sl] * scale
                return c

            lax.fori_loop(0, _CHUNK, scale_row, 0, unroll=2)
            pltpu.sync_copy(buf, out_hbm.at[pl.ds(base + j * _CHUNK, _CHUNK)])
            return carry

        lax.fori_loop(0, n_chunks, chunk_body, 0)

    return sc_gather


def kernel(input_ids, token_table):
    batch, seq_len = input_ids.shape
    d_model = token_table.shape[1]
    assert d_model == _D
    scale = float(d_model) ** 0.5

    ids = input_ids.reshape(-1).astype(jnp.int32)
    total_rows = ids.shape[0]
    num_workers = 32
    rows_per_worker = total_rows // num_workers
    n_chunks = rows_per_worker // _CHUNK
    ids3 = ids.reshape(num_workers, n_chunks, _CHUNK)

    out = _make_sc_gather(num_workers, n_chunks, rows_per_worker, total_rows, scale)(
        ids3, token_table
    )
    return out.reshape(batch, seq_len, d_model)
